# Initial kernel scaffold; baseline (speedup 1.0000x reference)
import jax
import jax.numpy as jnp
from jax.experimental import pallas as pl


def kernel(inputs, emb, W, b, gamma, beta):
    raise NotImplementedError("write your pallas kernel here")


# speedup vs baseline: 2.7440x; 2.7440x over previous
import functools

import jax
import jax.numpy as jnp
from jax import lax
from jax.experimental import pallas as pl
from jax.experimental.pallas import tpu as pltpu
from jax.experimental.pallas import tpu_sc as plsc

_NC = 2
_NS = 16
_NW = _NC * _NS

_CHUNK_IDX = 100
_NBUF = 4


def _sc_pool(idx3, emb, B, L, D):
    n_chunks = idx3.shape[1]
    rows_per_chunk = _CHUNK_IDX // L
    rows_per_w = n_chunks * rows_per_chunk
    nvec = D // 16
    inv_l = jnp.float32(1.0 / L)

    mesh = plsc.VectorSubcoreMesh(core_axis_name="c", subcore_axis_name="s")

    @functools.partial(
        pl.kernel,
        mesh=mesh,
        out_type=jax.ShapeDtypeStruct((B, D), jnp.float32),
        scratch_types=[
            pltpu.VMEM((n_chunks, _CHUNK_IDX), jnp.int32),
            pltpu.VMEM((_NBUF, _CHUNK_IDX, D), jnp.float32),
            pltpu.VMEM((rows_per_w, D), jnp.float32),
            pltpu.SemaphoreType.DMA((_NBUF,)),
        ],
        compiler_params=pltpu.CompilerParams(use_tc_tiling_on_sc=False),
    )
    def sc_kernel(idx_hbm, emb_hbm, out_hbm, idx_v, rows_v, pooled_v, sems):
        w = lax.axis_index("s") * _NC + lax.axis_index("c")
        pltpu.sync_copy(idx_hbm.at[w], idx_v)

        for kslot in range(_NBUF):
            pltpu.make_async_copy(
                emb_hbm.at[idx_v.at[kslot]], rows_v.at[kslot], sems.at[kslot]
            ).start()

        def outer(g, carry):
            for kslot in range(_NBUF):
                t = g * _NBUF + kslot
                pltpu.make_async_copy(
                    emb_hbm.at[idx_v.at[t]], rows_v.at[kslot], sems.at[kslot]
                ).wait()
                for cr in range(rows_per_chunk):
                    accs = tuple(jnp.zeros((16,), jnp.float32) for _ in range(nvec))

                    def inner(i5, a, _k=kslot, _cr=cr):
                        base = _cr * L + i5 * 5
                        for u in range(5):
                            a = tuple(
                                a[j] + rows_v[_k, base + u, pl.ds(j * 16, 16)]
                                for j in range(nvec)
                            )
                        return a

                    accs = lax.fori_loop(0, L // 5, inner, accs)
                    row = t * rows_per_chunk + cr
                    for j in range(nvec):
                        pooled_v[row, pl.ds(j * 16, 16)] = accs[j] * inv_l
                nxt = t + _NBUF

                @pl.when(nxt < n_chunks)
                def _():
                    pltpu.make_async_copy(
                        emb_hbm.at[idx_v.at[nxt]], rows_v.at[kslot], sems.at[kslot]
                    ).start()
            return carry

        lax.fori_loop(0, n_chunks // _NBUF, outer, jnp.int32(0))
        pltpu.sync_copy(pooled_v, out_hbm.at[pl.ds(w * rows_per_w, rows_per_w)])

    return sc_kernel(idx3, emb)


def _tc_head(pooled, W, b, gamma, beta):
    B, D = pooled.shape

    def body(x_ref, w_ref, b_ref, g_ref, bt_ref, o_ref):
        x = x_ref[...]
        y = lax.dot_general(
            x, w_ref[...], (((1,), (1,)), ((), ())),
            preferred_element_type=jnp.float32,
            precision=lax.Precision.HIGHEST,
        ) + b_ref[...]
        mu = jnp.mean(y, axis=0, keepdims=True)
        var = jnp.mean((y - mu) ** 2, axis=0, keepdims=True)
        yn = (y - mu) * lax.rsqrt(var + 1e-5) * g_ref[...] + bt_ref[...]
        inv_norm = lax.rsqrt(jnp.sum(yn * yn, axis=1, keepdims=True))
        o_ref[...] = yn * inv_norm

    return pl.pallas_call(
        body,
        out_shape=jax.ShapeDtypeStruct((B, D), jnp.float32),
    )(pooled, W, b.reshape(1, D), gamma.reshape(1, D), beta.reshape(1, D))


def kernel(inputs, emb, W, b, gamma, beta):
    B, L = inputs.shape
    D = emb.shape[1]
    idx = inputs.astype(jnp.int32)
    n_per_w = (B // _NW) * L
    n_chunks = n_per_w // _CHUNK_IDX
    idx3 = idx.reshape(_NW, n_chunks, _CHUNK_IDX)
    pooled = _sc_pool(idx3, emb, B, L, D)
    return _tc_head(pooled, W, b, gamma, beta)


# speedup vs baseline: 3.3880x; 1.2347x over previous
import functools

import jax
import jax.numpy as jnp
from jax import lax
from jax.experimental import pallas as pl
from jax.experimental.pallas import tpu as pltpu
from jax.experimental.pallas import tpu_sc as plsc

_NC = 2
_NS = 16
_NW = _NC * _NS

_CHUNK_IDX = 100
_NBUF = 4
def _tc_pair(embT, V, D):
    BLK = 2048
    nblk = -(-V // BLK)

    def body(x_ref, o_ref):
        xt = jnp.transpose(x_ref[...])
        o_ref[...] = jnp.concatenate([xt[: BLK // 2], xt[BLK // 2 :]], axis=1)

    return pl.pallas_call(
        body,
        grid=(nblk,),
        in_specs=[pl.BlockSpec((D, BLK), lambda i: (0, i))],
        out_specs=pl.BlockSpec((BLK // 2, 2 * D), lambda i: (i, 0)),
        out_shape=jax.ShapeDtypeStruct((nblk * BLK // 2, 2 * D), jnp.float32),
    )(embT)


def _sc_pool(idx3, emb, B, L, D):
    n_chunks = idx3.shape[1]
    rows_per_chunk = _CHUNK_IDX // L
    rows_per_w = n_chunks * rows_per_chunk
    nvec = D // 16
    inv_l = jnp.float32(1.0 / L)

    mesh = plsc.VectorSubcoreMesh(core_axis_name="c", subcore_axis_name="s")

    @functools.partial(
        pl.kernel,
        mesh=mesh,
        out_type=jax.ShapeDtypeStruct((B, D), jnp.float32),
        scratch_types=[
            pltpu.VMEM((n_chunks, _CHUNK_IDX), jnp.int32),
            pltpu.VMEM((_NBUF, _CHUNK_IDX, D), jnp.float32),
            pltpu.VMEM((rows_per_w, D), jnp.float32),
            pltpu.SemaphoreType.DMA((_NBUF,)),
        ],
        compiler_params=pltpu.CompilerParams(use_tc_tiling_on_sc=False),
    )
    def sc_kernel(idx_hbm, emb_hbm, out_hbm, idx_v, rows_v, pooled_v, sems):
        w = lax.axis_index("s") * _NC + lax.axis_index("c")
        pltpu.sync_copy(idx_hbm.at[w], idx_v)

        for kslot in range(_NBUF):
            pltpu.make_async_copy(
                emb_hbm.at[idx_v.at[kslot]], rows_v.at[kslot], sems.at[kslot]
            ).start()

        def outer(g, carry):
            for kslot in range(_NBUF):
                t = g * _NBUF + kslot
                pltpu.make_async_copy(
                    emb_hbm.at[idx_v.at[t]], rows_v.at[kslot], sems.at[kslot]
                ).wait()
                for cr in range(rows_per_chunk):
                    accs = tuple(jnp.zeros((16,), jnp.float32) for _ in range(nvec))

                    def inner(i5, a, _k=kslot, _cr=cr):
                        base = _cr * L + i5 * 5
                        for u in range(5):
                            a = tuple(
                                a[j] + rows_v[_k, base + u, pl.ds(j * 16, 16)]
                                for j in range(nvec)
                            )
                        return a

                    accs = lax.fori_loop(0, L // 5, inner, accs)
                    row = t * rows_per_chunk + cr
                    for j in range(nvec):
                        pooled_v[row, pl.ds(j * 16, 16)] = accs[j] * inv_l
                nxt = t + _NBUF

                @pl.when(nxt < n_chunks)
                def _():
                    pltpu.make_async_copy(
                        emb_hbm.at[idx_v.at[nxt]], rows_v.at[kslot], sems.at[kslot]
                    ).start()
            return carry

        lax.fori_loop(0, n_chunks // _NBUF, outer, jnp.int32(0))
        pltpu.sync_copy(pooled_v, out_hbm.at[pl.ds(w * rows_per_w, rows_per_w)])

    return sc_kernel(idx3, emb)


def _tc_head(pooled, W, b, gamma, beta):
    B, D = pooled.shape

    def body(x_ref, w_ref, b_ref, g_ref, bt_ref, o_ref):
        x = x_ref[...]
        y = lax.dot_general(
            x, w_ref[...], (((1,), (1,)), ((), ())),
            preferred_element_type=jnp.float32,
            precision=lax.Precision.HIGHEST,
        ) + b_ref[...]
        mu = jnp.mean(y, axis=0, keepdims=True)
        var = jnp.mean((y - mu) ** 2, axis=0, keepdims=True)
        yn = (y - mu) * lax.rsqrt(var + 1e-5) * g_ref[...] + bt_ref[...]
        inv_norm = lax.rsqrt(jnp.sum(yn * yn, axis=1, keepdims=True))
        o_ref[...] = yn * inv_norm

    return pl.pallas_call(
        body,
        out_shape=jax.ShapeDtypeStruct((B, D), jnp.float32),
    )(pooled, W, b.reshape(1, D), gamma.reshape(1, D), beta.reshape(1, D))


def kernel(inputs, emb, W, b, gamma, beta):
    B, L = inputs.shape
    D = emb.shape[1]
    V = emb.shape[0]
    idx = inputs.astype(jnp.int32)
    q = idx & 2047
    idx = (idx & ~jnp.int32(2047)) + ((q & 1023) << 1) + (q >> 10)
    n_per_w = (B // _NW) * L
    n_chunks = n_per_w // _CHUNK_IDX
    idx3 = idx.reshape(_NW, n_chunks, _CHUNK_IDX)
    embP = _tc_pair(emb.T, V, D)
    embL = embP.reshape(embP.shape[0] * 2, D)
    pooled = _sc_pool(idx3, embL, B, L, D)
    return _tc_head(pooled, W, b, gamma, beta)


# speedup vs baseline: 4.8872x; 1.4425x over previous
import functools

import jax
import jax.numpy as jnp
from jax import lax
from jax.experimental import pallas as pl
from jax.experimental.pallas import tpu as pltpu
from jax.experimental.pallas import tpu_sc as plsc

_NC = 2
_NS = 16
_NW = _NC * _NS

_CHUNK_IDX = 100
_NBUF = 4
def _tc_pair(embT, V, D):
    BLK = 8192
    nblk = -(-V // BLK)

    def body(x_ref, o_ref):
        xt = jnp.transpose(x_ref[...])
        o_ref[...] = jnp.concatenate([xt[: BLK // 2], xt[BLK // 2 :]], axis=1)

    return pl.pallas_call(
        body,
        grid=(nblk,),
        in_specs=[pl.BlockSpec((D, BLK), lambda i: (0, i))],
        out_specs=pl.BlockSpec((BLK // 2, 2 * D), lambda i: (i, 0)),
        out_shape=jax.ShapeDtypeStruct((nblk * BLK // 2, 2 * D), jnp.float32),
    )(embT)


def _sc_pool(idx3, emb, B, L, D):
    n_chunks = idx3.shape[1]
    rows_per_chunk = _CHUNK_IDX // L
    rows_per_w = n_chunks * rows_per_chunk
    nvec = D // 16
    inv_l = jnp.float32(1.0 / L)

    mesh = plsc.VectorSubcoreMesh(core_axis_name="c", subcore_axis_name="s")

    @functools.partial(
        pl.kernel,
        mesh=mesh,
        out_type=jax.ShapeDtypeStruct((B, D), jnp.float32),
        scratch_types=[
            pltpu.VMEM((n_chunks, _CHUNK_IDX), jnp.int32),
            pltpu.VMEM((_NBUF, _CHUNK_IDX, D), jnp.float32),
            pltpu.VMEM((rows_per_w, D), jnp.float32),
            pltpu.SemaphoreType.DMA((_NBUF,)),
        ],
        compiler_params=pltpu.CompilerParams(use_tc_tiling_on_sc=False),
    )
    def sc_kernel(idx_hbm, emb_hbm, out_hbm, idx_v, rows_v, pooled_v, sems):
        w = lax.axis_index("s") * _NC + lax.axis_index("c")
        pltpu.sync_copy(idx_hbm.at[w], idx_v)

        for kslot in range(_NBUF):
            pltpu.make_async_copy(
                emb_hbm.at[idx_v.at[kslot]], rows_v.at[kslot], sems.at[kslot]
            ).start()

        def outer(g, carry):
            for kslot in range(_NBUF):
                t = g * _NBUF + kslot
                pltpu.make_async_copy(
                    emb_hbm.at[idx_v.at[t]], rows_v.at[kslot], sems.at[kslot]
                ).wait()
                for cr in range(rows_per_chunk):
                    accs = tuple(jnp.zeros((16,), jnp.float32) for _ in range(nvec))

                    def inner(i5, a, _k=kslot, _cr=cr):
                        base = _cr * L + i5 * 5
                        for u in range(5):
                            a = tuple(
                                a[j] + rows_v[_k, base + u, pl.ds(j * 16, 16)]
                                for j in range(nvec)
                            )
                        return a

                    accs = lax.fori_loop(0, L // 5, inner, accs)
                    row = t * rows_per_chunk + cr
                    for j in range(nvec):
                        pooled_v[row, pl.ds(j * 16, 16)] = accs[j] * inv_l
                nxt = t + _NBUF

                @pl.when(nxt < n_chunks)
                def _():
                    pltpu.make_async_copy(
                        emb_hbm.at[idx_v.at[nxt]], rows_v.at[kslot], sems.at[kslot]
                    ).start()
            return carry

        lax.fori_loop(0, n_chunks // _NBUF, outer, jnp.int32(0))
        pltpu.sync_copy(pooled_v, out_hbm.at[pl.ds(w * rows_per_w, rows_per_w)])

    return sc_kernel(idx3, emb)


def _tc_head(pooled, W, b, gamma, beta):
    B, D = pooled.shape

    def body(x_ref, w_ref, b_ref, g_ref, bt_ref, o_ref):
        x = x_ref[...]
        y = lax.dot_general(
            x, w_ref[...], (((1,), (1,)), ((), ())),
            preferred_element_type=jnp.float32,
            precision=lax.Precision.HIGHEST,
        ) + b_ref[...]
        mu = jnp.mean(y, axis=0, keepdims=True)
        var = jnp.mean((y - mu) ** 2, axis=0, keepdims=True)
        yn = (y - mu) * lax.rsqrt(var + 1e-5) * g_ref[...] + bt_ref[...]
        inv_norm = lax.rsqrt(jnp.sum(yn * yn, axis=1, keepdims=True))
        o_ref[...] = yn * inv_norm

    return pl.pallas_call(
        body,
        out_shape=jax.ShapeDtypeStruct((B, D), jnp.float32),
    )(pooled, W, b.reshape(1, D), gamma.reshape(1, D), beta.reshape(1, D))


def kernel(inputs, emb, W, b, gamma, beta):
    B, L = inputs.shape
    D = emb.shape[1]
    V = emb.shape[0]
    idx = inputs.astype(jnp.int32)
    blk = 8192
    q = idx & (blk - 1)
    idx = (idx & ~jnp.int32(blk - 1)) + ((q & (blk // 2 - 1)) << 1) + (q >> 12)
    n_per_w = (B // _NW) * L
    n_chunks = n_per_w // _CHUNK_IDX
    idx3 = idx.reshape(_NW, n_chunks, _CHUNK_IDX)
    embP = _tc_pair(emb.T, V, D)
    embL = embP.reshape(embP.shape[0] * 2, D)
    pooled = _sc_pool(idx3, embL, B, L, D)
    return _tc_head(pooled, W, b, gamma, beta)


# speedup vs baseline: 5.2932x; 1.0831x over previous
import functools

import jax
import jax.numpy as jnp
from jax import lax
from jax.experimental import pallas as pl
from jax.experimental.pallas import tpu as pltpu
from jax.experimental.pallas import tpu_sc as plsc

_NC = 2
_NS = 16
_NW = _NC * _NS

_CHUNK_IDX = 100
_NBUF = 4
def _tc_pair(embT, V, D):
    BLK = 16384
    nblk = -(-V // BLK)

    def body(x_ref, o_ref):
        xt = jnp.transpose(x_ref[...])
        o_ref[...] = jnp.concatenate([xt[: BLK // 2], xt[BLK // 2 :]], axis=1)

    return pl.pallas_call(
        body,
        grid=(nblk,),
        in_specs=[pl.BlockSpec((D, BLK), lambda i: (0, i))],
        out_specs=pl.BlockSpec((BLK // 2, 2 * D), lambda i: (i, 0)),
        out_shape=jax.ShapeDtypeStruct((nblk * BLK // 2, 2 * D), jnp.float32),
    )(embT)


def _sc_pool(idx3, emb, B, L, D):
    n_chunks = idx3.shape[1]
    rows_per_chunk = _CHUNK_IDX // L
    rows_per_w = n_chunks * rows_per_chunk
    nvec = D // 16
    inv_l = jnp.float32(1.0 / L)

    mesh = plsc.VectorSubcoreMesh(core_axis_name="c", subcore_axis_name="s")

    @functools.partial(
        pl.kernel,
        mesh=mesh,
        out_type=jax.ShapeDtypeStruct((B, D), jnp.float32),
        scratch_types=[
            pltpu.VMEM((n_chunks, _CHUNK_IDX), jnp.int32),
            pltpu.VMEM((_NBUF, _CHUNK_IDX, D), jnp.float32),
            pltpu.VMEM((rows_per_w, D), jnp.float32),
            pltpu.SemaphoreType.DMA((_NBUF,)),
        ],
        compiler_params=pltpu.CompilerParams(use_tc_tiling_on_sc=False),
    )
    def sc_kernel(idx_hbm, emb_hbm, out_hbm, idx_v, rows_v, pooled_v, sems):
        w = lax.axis_index("s") * _NC + lax.axis_index("c")
        pltpu.sync_copy(idx_hbm.at[w], idx_v)

        for kslot in range(_NBUF):
            pltpu.make_async_copy(
                emb_hbm.at[idx_v.at[kslot]], rows_v.at[kslot], sems.at[kslot]
            ).start()

        def outer(g, carry):
            for kslot in range(_NBUF):
                t = g * _NBUF + kslot
                pltpu.make_async_copy(
                    emb_hbm.at[idx_v.at[t]], rows_v.at[kslot], sems.at[kslot]
                ).wait()
                for cr in range(rows_per_chunk):
                    accs = tuple(jnp.zeros((16,), jnp.float32) for _ in range(nvec))

                    def inner(i5, a, _k=kslot, _cr=cr):
                        base = _cr * L + i5 * 5
                        for u in range(5):
                            a = tuple(
                                a[j] + rows_v[_k, base + u, pl.ds(j * 16, 16)]
                                for j in range(nvec)
                            )
                        return a

                    accs = lax.fori_loop(0, L // 5, inner, accs)
                    row = t * rows_per_chunk + cr
                    for j in range(nvec):
                        pooled_v[row, pl.ds(j * 16, 16)] = accs[j] * inv_l
                nxt = t + _NBUF

                @pl.when(nxt < n_chunks)
                def _():
                    pltpu.make_async_copy(
                        emb_hbm.at[idx_v.at[nxt]], rows_v.at[kslot], sems.at[kslot]
                    ).start()
            return carry

        lax.fori_loop(0, n_chunks // _NBUF, outer, jnp.int32(0))
        pltpu.sync_copy(pooled_v, out_hbm.at[pl.ds(w * rows_per_w, rows_per_w)])

    return sc_kernel(idx3, emb)


def _tc_head(pooled, W, b, gamma, beta):
    B, D = pooled.shape

    def body(x_ref, w_ref, b_ref, g_ref, bt_ref, o_ref):
        x = x_ref[...]
        y = lax.dot_general(
            x, w_ref[...], (((1,), (1,)), ((), ())),
            preferred_element_type=jnp.float32,
            precision=lax.Precision.HIGHEST,
        ) + b_ref[...]
        mu = jnp.mean(y, axis=0, keepdims=True)
        var = jnp.mean((y - mu) ** 2, axis=0, keepdims=True)
        yn = (y - mu) * lax.rsqrt(var + 1e-5) * g_ref[...] + bt_ref[...]
        inv_norm = lax.rsqrt(jnp.sum(yn * yn, axis=1, keepdims=True))
        o_ref[...] = yn * inv_norm

    return pl.pallas_call(
        body,
        out_shape=jax.ShapeDtypeStruct((B, D), jnp.float32),
    )(pooled, W, b.reshape(1, D), gamma.reshape(1, D), beta.reshape(1, D))


def kernel(inputs, emb, W, b, gamma, beta):
    B, L = inputs.shape
    D = emb.shape[1]
    V = emb.shape[0]
    idx = inputs.astype(jnp.int32)
    blk = 16384
    q = idx & (blk - 1)
    idx = (idx & ~jnp.int32(blk - 1)) + ((q & (blk // 2 - 1)) << 1) + (q >> 13)
    n_per_w = (B // _NW) * L
    n_chunks = n_per_w // _CHUNK_IDX
    idx3 = idx.reshape(_NW, n_chunks, _CHUNK_IDX)
    embP = _tc_pair(emb.T, V, D)
    embL = embP.reshape(embP.shape[0] * 2, D)
    pooled = _sc_pool(idx3, embL, B, L, D)
    return _tc_head(pooled, W, b, gamma, beta)


# speedup vs baseline: 5.5187x; 1.0426x over previous
import functools

import jax
import jax.numpy as jnp
from jax import lax
from jax.experimental import pallas as pl
from jax.experimental.pallas import tpu as pltpu
from jax.experimental.pallas import tpu_sc as plsc

_NC = 2
_NS = 16
_NW = _NC * _NS

_CHUNK_IDX = 100
_NBUF = 4
def _tc_pair(embT, V, D):
    BLK = 32768
    nblk = -(-V // BLK)

    def body(x_ref, o_ref):
        xt = jnp.transpose(x_ref[...])
        o_ref[...] = jnp.concatenate([xt[: BLK // 2], xt[BLK // 2 :]], axis=1)

    return pl.pallas_call(
        body,
        grid=(nblk,),
        in_specs=[pl.BlockSpec((D, BLK), lambda i: (0, i))],
        out_specs=pl.BlockSpec((BLK // 2, 2 * D), lambda i: (i, 0)),
        out_shape=jax.ShapeDtypeStruct((nblk * BLK // 2, 2 * D), jnp.float32),
    )(embT)


def _sc_pool(idx3, emb, B, L, D):
    n_chunks = idx3.shape[1]
    rows_per_chunk = _CHUNK_IDX // L
    rows_per_w = n_chunks * rows_per_chunk
    nvec = D // 16
    inv_l = jnp.float32(1.0 / L)

    mesh = plsc.VectorSubcoreMesh(core_axis_name="c", subcore_axis_name="s")

    @functools.partial(
        pl.kernel,
        mesh=mesh,
        out_type=jax.ShapeDtypeStruct((B, D), jnp.float32),
        scratch_types=[
            pltpu.VMEM((n_chunks, _CHUNK_IDX), jnp.int32),
            pltpu.VMEM((_NBUF, _CHUNK_IDX, D), jnp.float32),
            pltpu.VMEM((rows_per_w, D), jnp.float32),
            pltpu.SemaphoreType.DMA((_NBUF,)),
        ],
        compiler_params=pltpu.CompilerParams(use_tc_tiling_on_sc=False),
    )
    def sc_kernel(idx_hbm, emb_hbm, out_hbm, idx_v, rows_v, pooled_v, sems):
        w = lax.axis_index("s") * _NC + lax.axis_index("c")
        pltpu.sync_copy(idx_hbm.at[w], idx_v)

        for kslot in range(_NBUF):
            pltpu.make_async_copy(
                emb_hbm.at[idx_v.at[kslot]], rows_v.at[kslot], sems.at[kslot]
            ).start()

        def outer(g, carry):
            for kslot in range(_NBUF):
                t = g * _NBUF + kslot
                pltpu.make_async_copy(
                    emb_hbm.at[idx_v.at[t]], rows_v.at[kslot], sems.at[kslot]
                ).wait()
                for cr in range(rows_per_chunk):
                    accs = tuple(jnp.zeros((16,), jnp.float32) for _ in range(nvec))

                    def inner(i5, a, _k=kslot, _cr=cr):
                        base = _cr * L + i5 * 5
                        for u in range(5):
                            a = tuple(
                                a[j] + rows_v[_k, base + u, pl.ds(j * 16, 16)]
                                for j in range(nvec)
                            )
                        return a

                    accs = lax.fori_loop(0, L // 5, inner, accs)
                    row = t * rows_per_chunk + cr
                    for j in range(nvec):
                        pooled_v[row, pl.ds(j * 16, 16)] = accs[j] * inv_l
                nxt = t + _NBUF

                @pl.when(nxt < n_chunks)
                def _():
                    pltpu.make_async_copy(
                        emb_hbm.at[idx_v.at[nxt]], rows_v.at[kslot], sems.at[kslot]
                    ).start()
            return carry

        lax.fori_loop(0, n_chunks // _NBUF, outer, jnp.int32(0))
        pltpu.sync_copy(pooled_v, out_hbm.at[pl.ds(w * rows_per_w, rows_per_w)])

    return sc_kernel(idx3, emb)


def _tc_head(pooled, W, b, gamma, beta):
    B, D = pooled.shape

    def body(x_ref, w_ref, b_ref, g_ref, bt_ref, o_ref):
        x = x_ref[...]
        y = lax.dot_general(
            x, w_ref[...], (((1,), (1,)), ((), ())),
            preferred_element_type=jnp.float32,
            precision=lax.Precision.HIGHEST,
        ) + b_ref[...]
        mu = jnp.mean(y, axis=0, keepdims=True)
        var = jnp.mean((y - mu) ** 2, axis=0, keepdims=True)
        yn = (y - mu) * lax.rsqrt(var + 1e-5) * g_ref[...] + bt_ref[...]
        inv_norm = lax.rsqrt(jnp.sum(yn * yn, axis=1, keepdims=True))
        o_ref[...] = yn * inv_norm

    return pl.pallas_call(
        body,
        out_shape=jax.ShapeDtypeStruct((B, D), jnp.float32),
    )(pooled, W, b.reshape(1, D), gamma.reshape(1, D), beta.reshape(1, D))


def kernel(inputs, emb, W, b, gamma, beta):
    B, L = inputs.shape
    D = emb.shape[1]
    V = emb.shape[0]
    idx = inputs.astype(jnp.int32)
    blk = 32768
    q = idx & (blk - 1)
    idx = (idx & ~jnp.int32(blk - 1)) + ((q & (blk // 2 - 1)) << 1) + (q >> 14)
    n_per_w = (B // _NW) * L
    n_chunks = n_per_w // _CHUNK_IDX
    idx3 = idx.reshape(_NW, n_chunks, _CHUNK_IDX)
    embP = _tc_pair(emb.T, V, D)
    embL = embP.reshape(embP.shape[0] * 2, D)
    pooled = _sc_pool(idx3, embL, B, L, D)
    return _tc_head(pooled, W, b, gamma, beta)
